# baseline (speedup 1.0000x reference)
import functools

import jax
import jax.numpy as jnp
from jax import lax
from jax.experimental import pallas as pl
from jax.experimental.pallas import tpu as pltpu
from jax.experimental.pallas import tpu_sc as plsc

OUT_DIM = 1000
BATCH = 16384
CHANNEL = 512
HALF = CHANNEL // 2
LANES = 16

NUM_WORKERS = 32
ROWS_PER_WORKER = BATCH // NUM_WORKERS
R = 16
NUM_CHUNKS = ROWS_PER_WORKER // R
NB = 2
NUM_GROUPS = NUM_CHUNKS // NB


def _build():
    mesh = plsc.VectorSubcoreMesh(core_axis_name="c", subcore_axis_name="s")

    @functools.partial(
        pl.kernel,
        mesh=mesh,
        out_type=jax.ShapeDtypeStruct((BATCH, OUT_DIM), jnp.float32),
        scratch_types=[
            pltpu.VMEM((R, CHANNEL), jnp.float32),
            pltpu.VMEM((R, CHANNEL), jnp.float32),
            pltpu.VMEM((R, HALF), jnp.int32),
            pltpu.VMEM((R, HALF), jnp.int32),
            pltpu.VMEM((R, OUT_DIM), jnp.float32),
            pltpu.VMEM((R, OUT_DIM), jnp.float32),
            pltpu.SemaphoreType.DMA,
            pltpu.SemaphoreType.DMA,
            pltpu.SemaphoreType.DMA,
            pltpu.SemaphoreType.DMA,
        ],
        compiler_params=pltpu.CompilerParams(
            needs_layout_passes=False,
            disable_bounds_checks=True,
            disable_semaphore_checks=True,
        ),
    )
    def run(
        dec_hbm, cid_hbm, out_hbm,
        dec_v0, dec_v1, cid_v0, cid_v1, out_v0, out_v1,
        si0, si1, so0, so1,
    ):
        cid = lax.axis_index("c")
        sid = lax.axis_index("s")
        wid = sid * 2 + cid
        row0 = wid * ROWS_PER_WORKER

        dec_v = (dec_v0, dec_v1)
        cid_v = (cid_v0, cid_v1)
        out_v = (out_v0, out_v1)
        sem_in = (si0, si1)
        sem_out = (so0, so1)
        zeros = jnp.zeros((LANES,), jnp.float32)

        def in_desc(ci, b):
            base = row0 + ci * R
            return (
                pltpu.make_async_copy(
                    dec_hbm.at[pl.ds(base, R)], dec_v[b], sem_in[b]
                ),
                pltpu.make_async_copy(
                    cid_hbm.at[pl.ds(base, R)], cid_v[b], sem_in[b]
                ),
            )

        def out_desc(ci, b):
            base = row0 + ci * R
            return pltpu.make_async_copy(
                out_v[b], out_hbm.at[pl.ds(base, R)], sem_out[b]
            )

        zero_sites = [
            (rr, min(jj * LANES, OUT_DIM - LANES))
            for rr in range(R)
            for jj in range(-(-OUT_DIM // LANES))
        ]
        Z_PER_ITER = -(-len(zero_sites) // (R * (HALF // LANES)))

        def zero_full(b):
            def zero_body(r, _):
                for j in range(OUT_DIM // LANES):
                    out_v[b][r, pl.ds(j * LANES, LANES)] = zeros
                out_v[b][r, pl.ds(OUT_DIM - LANES, LANES)] = zeros
                return ()

            lax.fori_loop(0, R, zero_body, ())

        for b in range(NB):
            d0, d1 = in_desc(b, b)
            d0.start()
            d1.start()
        zero_full(0)

        def group_body(g, _):
            for b in range(NB):
                bn = (b + 1) % NB
                ci = g * NB + b
                d0, d1 = in_desc(ci, b)
                d0.wait()
                d1.wait()

                @pl.when(ci + 1 - NB >= 0)
                def _():
                    out_desc(ci + 1 - NB, bn).wait()

                zi = 0
                for r in range(R):
                    rvec = jnp.full((LANES,), r, jnp.int32)
                    for k in range(HALF // LANES):
                        ids = cid_v[b][r, pl.ds(k * LANES, LANES)]
                        a = dec_v[b][r, pl.ds(k * LANES, LANES)]
                        c2 = dec_v[b][r, pl.ds(HALF + k * LANES, LANES)]
                        plsc.addupdate_scatter(
                            out_v[b], [rvec, ids], a + c2
                        )
                        for _ in range(Z_PER_ITER):
                            if zi < len(zero_sites):
                                zr, zc = zero_sites[zi]
                                zi += 1
                                out_v[bn][zr, pl.ds(zc, LANES)] = zeros

                out_desc(ci, b).start()

                @pl.when(g < NUM_GROUPS - 1)
                def _():
                    n0, n1 = in_desc(ci + NB, b)
                    n0.start()
                    n1.start()

            return ()

        lax.fori_loop(0, NUM_GROUPS, group_body, ())

        out_desc(NUM_CHUNKS - 1, (NUM_CHUNKS - 1) % NB).wait()

    return run


_RUN = _build()


@jax.jit
def kernel(decoded, class_id):
    out = _RUN(decoded, class_id.astype(jnp.int32))
    return out


# speedup vs baseline: 1.2288x; 1.2288x over previous
import functools

import jax
import jax.numpy as jnp
from jax import lax
from jax.experimental import pallas as pl
from jax.experimental.pallas import tpu as pltpu
from jax.experimental.pallas import tpu_sc as plsc

OUT_DIM = 1000
BATCH = 16384
CHANNEL = 512
HALF = CHANNEL // 2
LANES = 16
KVECS = HALF // LANES

NUM_WORKERS = 32
SW = 128
STRIPES_PER_WORKER = BATCH // (NUM_WORKERS * SW)
DEPTH = 3
NBANDS = 8
BAND = 128
BAND_SIZES = [BAND] * (NBANDS - 1) + [OUT_DIM - BAND * (NBANDS - 1)]


def _build():
    mesh = plsc.VectorSubcoreMesh(core_axis_name="c", subcore_axis_name="s")

    @functools.partial(
        pl.kernel,
        mesh=mesh,
        out_type=jax.ShapeDtypeStruct((OUT_DIM, BATCH), jnp.float32),
        scratch_types=[
            pltpu.VMEM((1, CHANNEL), jnp.float32),
            pltpu.VMEM((1, CHANNEL), jnp.float32),
            pltpu.VMEM((1, CHANNEL), jnp.float32),
            pltpu.VMEM((1, HALF), jnp.int32),
            pltpu.VMEM((1, HALF), jnp.int32),
            pltpu.VMEM((1, HALF), jnp.int32),
            pltpu.VMEM((OUT_DIM, SW), jnp.float32),
            pltpu.SemaphoreType.DMA,
            pltpu.SemaphoreType.DMA,
            pltpu.SemaphoreType.DMA,
            pltpu.SemaphoreType.DMA,
            pltpu.SemaphoreType.DMA,
            pltpu.SemaphoreType.DMA,
            pltpu.SemaphoreType.DMA,
            pltpu.SemaphoreType.DMA,
            pltpu.SemaphoreType.DMA,
            pltpu.SemaphoreType.DMA,
            pltpu.SemaphoreType.DMA,
        ],
        compiler_params=pltpu.CompilerParams(
            needs_layout_passes=False,
            disable_bounds_checks=True,
            disable_semaphore_checks=True,
        ),
    )
    def run(
        dec_hbm, cid_hbm, out_hbm,
        d0, d1, d2, c0, c1, c2, ob,
        si0, si1, si2,
        sb0, sb1, sb2, sb3, sb4, sb5, sb6, sb7,
    ):
        wid = lax.axis_index("s") * 2 + lax.axis_index("c")
        dec_v = (d0, d1, d2)
        cid_v = (c0, c1, c2)
        sem_in = (si0, si1, si2)
        sem_band = (sb0, sb1, sb2, sb3, sb4, sb5, sb6, sb7)
        zeros = jnp.zeros((LANES,), jnp.float32)

        def in_descs(row, t):
            return (
                pltpu.make_async_copy(
                    dec_hbm.at[pl.ds(row, 1)], dec_v[t], sem_in[t]
                ),
                pltpu.make_async_copy(
                    cid_hbm.at[pl.ds(row, 1)], cid_v[t], sem_in[t]
                ),
            )

        def start_in(row, t):
            a, b = in_descs(row, t)
            a.start()
            b.start()

        def wait_in(row, t):
            a, b = in_descs(row, t)
            a.wait()
            b.wait()

        def band_desc(k, col0):
            return pltpu.make_async_copy(
                ob.at[pl.ds(k * BAND, BAND_SIZES[k])],
                out_hbm.at[pl.ds(k * BAND, BAND_SIZES[k]), pl.ds(col0, SW)],
                sem_band[k],
            )

        def zero_band(k):
            def body(c, _):
                for j in range(SW // LANES):
                    ob[k * BAND + c, pl.ds(j * LANES, LANES)] = zeros
                return ()

            lax.fori_loop(0, BAND_SIZES[k], body, ())

        def scatter_row(r, t):
            colv = jnp.zeros((LANES,), jnp.int32) + r
            for k in range(KVECS):
                ids = cid_v[t][0, pl.ds(k * LANES, LANES)]
                a = dec_v[t][0, pl.ds(k * LANES, LANES)]
                b2 = dec_v[t][0, pl.ds(HALF + k * LANES, LANES)]
                plsc.addupdate_scatter(ob, [ids, colv], a + b2)

        for k in range(NBANDS):
            zero_band(k)

        def stripe_body(s, _):
            col0 = pl.multiple_of((wid * STRIPES_PER_WORKER + s) * SW, SW)

            for t in range(DEPTH):
                start_in(col0 + t, t)

            def triple(g, _):
                for t in range(DEPTH):
                    r = DEPTH * g + t
                    wait_in(col0 + r, t)
                    scatter_row(r, t)
                    start_in(col0 + r + DEPTH, t)
                return ()

            lax.fori_loop(0, (SW - DEPTH * 2 + 1) // DEPTH, triple, ())

            base = SW - 5
            for i, t in enumerate((0, 1, 2, 0, 1)):
                r = base + i
                wait_in(col0 + r, t)
                scatter_row(r, t)
                if i < 2:
                    start_in(col0 + r + DEPTH, t)

            for k in range(NBANDS):
                band_desc(k, col0).start()
            for k in range(NBANDS):
                band_desc(k, col0).wait()
                zero_band(k)

            return ()

        lax.fori_loop(0, STRIPES_PER_WORKER, stripe_body, ())

    return run


_RUN = _build()


@jax.jit
def kernel(decoded, class_id):
    out_t = _RUN(decoded, class_id.astype(jnp.int32))
    return out_t.T


# speedup vs baseline: 1.3846x; 1.1268x over previous
import functools

import jax
import jax.numpy as jnp
from jax import lax
from jax.experimental import pallas as pl
from jax.experimental.pallas import tpu as pltpu
from jax.experimental.pallas import tpu_sc as plsc

OUT_DIM = 1000
BATCH = 16384
CHANNEL = 512
HALF = CHANNEL // 2
LANES = 16
KVECS = HALF // LANES

NUM_WORKERS = 32
SW = 128
STRIPES_PER_WORKER = BATCH // (NUM_WORKERS * SW)
DEPTH = 4
NBANDS = 8
BAND = 128
BAND_SIZES = [BAND] * (NBANDS - 1) + [OUT_DIM - BAND * (NBANDS - 1)]


def _build():
    mesh = plsc.VectorSubcoreMesh(core_axis_name="c", subcore_axis_name="s")

    @functools.partial(
        pl.kernel,
        mesh=mesh,
        out_type=jax.ShapeDtypeStruct((OUT_DIM, BATCH), jnp.float32),
        scratch_types=[
            pltpu.VMEM((1, CHANNEL), jnp.float32),
            pltpu.VMEM((1, CHANNEL), jnp.float32),
            pltpu.VMEM((1, CHANNEL), jnp.float32),
            pltpu.VMEM((1, CHANNEL), jnp.float32),
            pltpu.VMEM((1, HALF), jnp.int32),
            pltpu.VMEM((1, HALF), jnp.int32),
            pltpu.VMEM((1, HALF), jnp.int32),
            pltpu.VMEM((1, HALF), jnp.int32),
            pltpu.VMEM((OUT_DIM, SW), jnp.float32),
            pltpu.SemaphoreType.DMA,
            pltpu.SemaphoreType.DMA,
            pltpu.SemaphoreType.DMA,
            pltpu.SemaphoreType.DMA,
            pltpu.SemaphoreType.DMA,
            pltpu.SemaphoreType.DMA,
            pltpu.SemaphoreType.DMA,
            pltpu.SemaphoreType.DMA,
            pltpu.SemaphoreType.DMA,
            pltpu.SemaphoreType.DMA,
            pltpu.SemaphoreType.DMA,
            pltpu.SemaphoreType.DMA,
        ],
        compiler_params=pltpu.CompilerParams(
            needs_layout_passes=False,
            disable_bounds_checks=True,
            disable_semaphore_checks=True,
        ),
    )
    def run(
        dec_hbm, cid_hbm, out_hbm,
        d0, d1, d2, d3, c0, c1, c2, c3, ob,
        si0, si1, si2, si3,
        sb0, sb1, sb2, sb3, sb4, sb5, sb6, sb7,
    ):
        wid = lax.axis_index("s") * 2 + lax.axis_index("c")
        dec_v = (d0, d1, d2, d3)
        cid_v = (c0, c1, c2, c3)
        sem_in = (si0, si1, si2, si3)
        sem_band = (sb0, sb1, sb2, sb3, sb4, sb5, sb6, sb7)
        zeros = jnp.zeros((LANES,), jnp.float32)

        def in_descs(row, t):
            return (
                pltpu.make_async_copy(
                    dec_hbm.at[pl.ds(row, 1)], dec_v[t], sem_in[t]
                ),
                pltpu.make_async_copy(
                    cid_hbm.at[pl.ds(row, 1)], cid_v[t], sem_in[t]
                ),
            )

        def start_in(row, t):
            a, b = in_descs(row, t)
            a.start()
            b.start()

        def wait_in(row, t):
            a, b = in_descs(row, t)
            a.wait()
            b.wait()

        def band_desc(k, col0):
            return pltpu.make_async_copy(
                ob.at[pl.ds(k * BAND, BAND_SIZES[k])],
                out_hbm.at[pl.ds(k * BAND, BAND_SIZES[k]), pl.ds(col0, SW)],
                sem_band[k],
            )

        def zero_band(k):
            def body(c, _):
                for j in range(SW // LANES):
                    ob[k * BAND + c, pl.ds(j * LANES, LANES)] = zeros
                return ()

            lax.fori_loop(0, BAND_SIZES[k], body, (), unroll=4)

        def scatter_row(r, t):
            colv = jnp.zeros((LANES,), jnp.int32) + r
            for k in range(KVECS):
                ids = cid_v[t][0, pl.ds(k * LANES, LANES)]
                a = dec_v[t][0, pl.ds(k * LANES, LANES)]
                b2 = dec_v[t][0, pl.ds(HALF + k * LANES, LANES)]
                plsc.addupdate_scatter(ob, [ids, colv], a + b2)

        for k in range(NBANDS):
            zero_band(k)
        col_base = pl.multiple_of(wid * STRIPES_PER_WORKER * SW, SW)
        for t in range(DEPTH):
            start_in(col_base + t, t)

        def stripe_body(s, _):
            col0 = pl.multiple_of(col_base + s * SW, SW)

            def quad(g, _):
                for t in range(DEPTH):
                    r = DEPTH * g + t
                    wait_in(col0 + r, t)
                    scatter_row(r, t)
                    start_in(col0 + r + DEPTH, t)
                return ()

            lax.fori_loop(0, SW // DEPTH - 2, quad, ())

            base = SW - 2 * DEPTH
            for i in range(2 * DEPTH):
                t = i % DEPTH
                r = base + i
                wait_in(col0 + r, t)
                scatter_row(r, t)
                if i < DEPTH:
                    start_in(col0 + r + DEPTH, t)

            @pl.when(s < STRIPES_PER_WORKER - 1)
            def _():
                for t in range(DEPTH):
                    start_in(col0 + SW + t, t)

            for k in range(NBANDS):
                band_desc(k, col0).start()
            for k in range(NBANDS):
                band_desc(k, col0).wait()
                zero_band(k)

            return ()

        lax.fori_loop(0, STRIPES_PER_WORKER, stripe_body, ())

    return run


_RUN = _build()


@jax.jit
def kernel(decoded, class_id):
    out_t = _RUN(decoded, class_id.astype(jnp.int32))
    return out_t.T
